# Initial kernel scaffold; baseline (speedup 1.0000x reference)
import jax
import jax.numpy as jnp
from jax.experimental import pallas as pl


def kernel(t, pe):
    raise NotImplementedError("write your pallas kernel here")


# speedup vs baseline: 1.6319x; 1.6319x over previous
import functools

import jax
import jax.numpy as jnp
from jax import lax
from jax.experimental import pallas as pl
from jax.experimental.pallas import tpu as pltpu
from jax.experimental.pallas import tpu_sc as plsc

DIM = 1024
B = 16384
NC = 2
NS = 16
NW = NC * NS
B_PER_W = B // NW
CHUNK = 32
NCHUNK = B_PER_W // CHUNK


def _make_gather():
    mesh = plsc.VectorSubcoreMesh(core_axis_name="c", subcore_axis_name="s")

    @functools.partial(
        pl.kernel,
        mesh=mesh,
        out_type=jax.ShapeDtypeStruct((B, DIM), jnp.float32),
        scratch_types=[
            pltpu.VMEM((NCHUNK, CHUNK), jnp.int32),
            pltpu.VMEM((CHUNK, DIM), jnp.float32),
            pltpu.VMEM((CHUNK, DIM), jnp.float32),
            pltpu.SemaphoreType.DMA,
            pltpu.SemaphoreType.DMA,
            pltpu.SemaphoreType.DMA,
            pltpu.SemaphoreType.DMA,
        ],
    )
    def gather_kernel(idx_hbm, pe_hbm, out_hbm, idx_v, buf0, buf1,
                      g0, g1, w0, w1):
        wid = lax.axis_index("s") * NC + lax.axis_index("c")
        base = wid * B_PER_W
        pltpu.sync_copy(idx_hbm.at[wid], idx_v)

        bufs = (buf0, buf1)
        gsems = (g0, g1)
        wsems = (w0, w1)
        g_desc = [
            pltpu.async_copy(pe_hbm.at[idx_v.at[0]], buf0, g0),
            pltpu.async_copy(pe_hbm.at[idx_v.at[1]], buf1, g1),
        ]
        w_desc = [None, None]
        for j in range(NCHUNK):
            b = j % 2
            g_desc[b].wait()
            w_desc[b] = pltpu.async_copy(
                bufs[b], out_hbm.at[pl.ds(base + j * CHUNK, CHUNK)], wsems[b])
            nj = j + 2
            if nj < NCHUNK:
                w_desc[b].wait()
                g_desc[b] = pltpu.async_copy(
                    pe_hbm.at[idx_v.at[nj]], bufs[b], gsems[b])
        w_desc[0].wait()
        w_desc[1].wait()

    return gather_kernel


_GATHER = _make_gather()


def kernel(t, pe):
    idx = t.astype(jnp.int32).reshape(NW, NCHUNK, CHUNK)
    return _GATHER(idx, pe)


# speedup vs baseline: 1.6446x; 1.0078x over previous
import functools

import jax
import jax.numpy as jnp
from jax import lax
from jax.experimental import pallas as pl
from jax.experimental.pallas import tpu as pltpu
from jax.experimental.pallas import tpu_sc as plsc

DIM = 1024
B = 16384
NC = 2
NS = 16
NW = NC * NS
B_PER_W = B // NW
CHUNK = 32
NCHUNK = B_PER_W // CHUNK
NBUF = 3


def _make_gather():
    mesh = plsc.VectorSubcoreMesh(core_axis_name="c", subcore_axis_name="s")

    @functools.partial(
        pl.kernel,
        mesh=mesh,
        out_type=jax.ShapeDtypeStruct((B, DIM), jnp.float32),
        scratch_types=[
            pltpu.VMEM((NCHUNK, CHUNK), jnp.int32),
        ] + [pltpu.VMEM((CHUNK, DIM), jnp.float32) for _ in range(NBUF)]
          + [pltpu.SemaphoreType.DMA for _ in range(2 * NBUF)],
    )
    def gather_kernel(idx_hbm, pe_hbm, out_hbm, idx_v, *scratch):
        bufs = scratch[:NBUF]
        gsems = scratch[NBUF:2 * NBUF]
        wsems = scratch[2 * NBUF:]
        wid = lax.axis_index("s") * NC + lax.axis_index("c")
        base = wid * B_PER_W
        pltpu.sync_copy(idx_hbm.at[wid], idx_v)

        g_desc = [
            pltpu.async_copy(pe_hbm.at[idx_v.at[b]], bufs[b], gsems[b])
            for b in range(NBUF)
        ]
        w_desc = [None] * NBUF
        for j in range(NCHUNK):
            b = j % NBUF
            nj = j + NBUF - 1
            if NBUF <= nj < NCHUNK:
                nb = nj % NBUF
                w_desc[nb].wait()
                g_desc[nb] = pltpu.async_copy(
                    pe_hbm.at[idx_v.at[nj]], bufs[nb], gsems[nb])
            g_desc[b].wait()
            w_desc[b] = pltpu.async_copy(
                bufs[b], out_hbm.at[pl.ds(base + j * CHUNK, CHUNK)], wsems[b])
        for b in range(NBUF):
            w_desc[b].wait()

    return gather_kernel


_GATHER = _make_gather()


def kernel(t, pe):
    idx = t.astype(jnp.int32).reshape(NW, NCHUNK, CHUNK)
    return _GATHER(idx, pe)


# speedup vs baseline: 1.6532x; 1.0053x over previous
import functools

import jax
import jax.numpy as jnp
from jax import lax
from jax.experimental import pallas as pl
from jax.experimental.pallas import tpu as pltpu
from jax.experimental.pallas import tpu_sc as plsc

DIM = 1024
B = 16384
NC = 2
NS = 16
NW = NC * NS
B_PER_W = B // NW
CHUNK = 16
NCHUNK = B_PER_W // CHUNK
NBUF = 6


def _make_gather():
    mesh = plsc.VectorSubcoreMesh(core_axis_name="c", subcore_axis_name="s")

    @functools.partial(
        pl.kernel,
        mesh=mesh,
        out_type=jax.ShapeDtypeStruct((B, DIM), jnp.float32),
        scratch_types=[
            pltpu.VMEM((NCHUNK, CHUNK), jnp.int32),
        ] + [pltpu.VMEM((CHUNK, DIM), jnp.float32) for _ in range(NBUF)]
          + [pltpu.SemaphoreType.DMA for _ in range(2 * NBUF)],
    )
    def gather_kernel(idx_hbm, pe_hbm, out_hbm, idx_v, *scratch):
        bufs = scratch[:NBUF]
        gsems = scratch[NBUF:2 * NBUF]
        wsems = scratch[2 * NBUF:]
        wid = lax.axis_index("s") * NC + lax.axis_index("c")
        base = wid * B_PER_W
        pltpu.sync_copy(idx_hbm.at[wid], idx_v)

        g_desc = [
            pltpu.async_copy(pe_hbm.at[idx_v.at[b]], bufs[b], gsems[b])
            for b in range(NBUF)
        ]
        w_desc = [None] * NBUF
        for j in range(NCHUNK):
            b = j % NBUF
            nj = j + NBUF - 1
            if NBUF <= nj < NCHUNK:
                nb = nj % NBUF
                w_desc[nb].wait()
                g_desc[nb] = pltpu.async_copy(
                    pe_hbm.at[idx_v.at[nj]], bufs[nb], gsems[nb])
            g_desc[b].wait()
            w_desc[b] = pltpu.async_copy(
                bufs[b], out_hbm.at[pl.ds(base + j * CHUNK, CHUNK)], wsems[b])
        for b in range(NBUF):
            w_desc[b].wait()

    return gather_kernel


_GATHER = _make_gather()


def kernel(t, pe):
    idx = t.astype(jnp.int32).reshape(NW, NCHUNK, CHUNK)
    return _GATHER(idx, pe)


# speedup vs baseline: 1.6663x; 1.0079x over previous
import functools

import jax
import jax.numpy as jnp
from jax import lax
from jax.experimental import pallas as pl
from jax.experimental.pallas import tpu as pltpu
from jax.experimental.pallas import tpu_sc as plsc

DIM = 1024
B = 16384
NC = 2
NS = 16
NW = NC * NS
B_PER_W = B // NW
CHUNK = 16
NCHUNK = B_PER_W // CHUNK
NBUF = 6


def _make_gather():
    mesh = plsc.VectorSubcoreMesh(core_axis_name="c", subcore_axis_name="s")

    @functools.partial(
        pl.kernel,
        mesh=mesh,
        out_type=jax.ShapeDtypeStruct((B, DIM), jnp.float32),
        scratch_types=[
            pltpu.VMEM((B_PER_W,), jnp.int32),
        ] + [pltpu.VMEM((CHUNK, DIM), jnp.float32) for _ in range(NBUF)]
          + [pltpu.SemaphoreType.DMA for _ in range(2 * NBUF)],
    )
    def gather_kernel(idx_hbm, pe_hbm, out_hbm, idx_v, *scratch):
        bufs = scratch[:NBUF]
        gsems = scratch[NBUF:2 * NBUF]
        wsems = scratch[2 * NBUF:]
        wid = lax.axis_index("s") * NC + lax.axis_index("c")
        base = wid * B_PER_W
        pltpu.sync_copy(idx_hbm.at[pl.ds(base, B_PER_W)], idx_v)

        g_desc = [
            pltpu.async_copy(
                pe_hbm.at[idx_v.at[pl.ds(b * CHUNK, CHUNK)]], bufs[b],
                gsems[b])
            for b in range(NBUF)
        ]
        w_desc = [None] * NBUF
        for j in range(NCHUNK):
            b = j % NBUF
            nj = j + NBUF - 1
            if NBUF <= nj < NCHUNK:
                nb = nj % NBUF
                w_desc[nb].wait()
                g_desc[nb] = pltpu.async_copy(
                    pe_hbm.at[idx_v.at[pl.ds(nj * CHUNK, CHUNK)]], bufs[nb],
                    gsems[nb])
            g_desc[b].wait()
            w_desc[b] = pltpu.async_copy(
                bufs[b], out_hbm.at[pl.ds(base + j * CHUNK, CHUNK)], wsems[b])
        for b in range(NBUF):
            w_desc[b].wait()

    return gather_kernel


_GATHER = _make_gather()


def kernel(t, pe):
    return _GATHER(t.astype(jnp.int32), pe)
